# baseline (speedup 1.0000x reference)
import functools

import jax
import jax.numpy as jnp
from jax import lax
from jax.experimental import pallas as pl
from jax.experimental.pallas import tpu as pltpu
from jax.experimental.pallas import tpu_sc as plsc

N = 10000
E = 320000
NPAD = 10240

NUM_SUBCORES = 16
CHUNK = 128
TOT_CHUNKS = E // CHUNK
NCHUNK = 160
NB = 5
NGROUP = NCHUNK // NB
NSLOT = 4
HBM_SLOTS = ()
DEG_SLOTS_0 = (0, 1)
DEG_SLOTS_1 = (2, 3, 4)
RPT = NPAD // 16


def _make_sc_aggregate(D):
  half = D // 2
  mesh = plsc.VectorSubcoreMesh(core_axis_name="c", subcore_axis_name="s")

  @functools.partial(
      pl.kernel,
      out_type=(
          jax.ShapeDtypeStruct((NPAD, D), jnp.float32),
          jax.ShapeDtypeStruct((2, NPAD), jnp.float32),
      ),
      mesh=mesh,
      compiler_params=pltpu.CompilerParams(use_tc_tiling_on_sc=False),
      scratch_types=[
          pltpu.VMEM((NSLOT, NB, CHUNK), jnp.int32),
          pltpu.VMEM((NSLOT, NB, CHUNK), jnp.int32),
          pltpu.VMEM((NB, CHUNK, half), jnp.float32),
          pltpu.VMEM((CHUNK,), jnp.float32),
          pltpu.VMEM((RPT,), jnp.float32),
          pltpu.VMEM_SHARED((NPAD, half), jnp.float32),
          pltpu.VMEM_SHARED((NPAD, half), jnp.float32),
          pltpu.VMEM_SHARED((NPAD,), jnp.float32),
      ] + [pltpu.SemaphoreType.DMA] * (3 * NB + 2 * NSLOT),
  )
  def agg_kernel(pl_hbm, pr_hbm, src_hbm, dst_hbm, agg_hbm, deg_hbm,
                 sidx_v, didx_v, rows_v, ones_v, zdeg_v, p_sh, agg_sh, deg_sh,
                 *sems):
    gsem = sems[:NB]
    ssem = sems[NB:2 * NB]
    dsem = sems[2 * NB:3 * NB]
    isems = sems[3 * NB:3 * NB + NSLOT]
    isemd = sems[3 * NB + NSLOT:]
    cid = lax.axis_index("c")
    sid = lax.axis_index("s")
    ngrp = jnp.minimum(NCHUNK, TOT_CHUNKS - sid * NCHUNK) // NB

    zeros16 = jnp.zeros((16,), jnp.float32)
    ones16 = jnp.ones((16,), jnp.float32)

    pltpu.async_copy(src_hbm.at[pl.ds(sid * NCHUNK, NB)], sidx_v.at[0],
                     isems[0])
    pltpu.async_copy(dst_hbm.at[pl.ds(sid * NCHUNK, NB)], didx_v.at[0],
                     isemd[0])
    @pl.when(cid == 0)
    def _stage0():
      pltpu.sync_copy(pl_hbm.at[pl.ds(sid * RPT, RPT)],
                      p_sh.at[pl.ds(sid * RPT, RPT)])
    @pl.when(cid == 1)
    def _stage1():
      pltpu.sync_copy(pr_hbm.at[pl.ds(sid * RPT, RPT)],
                      p_sh.at[pl.ds(sid * RPT, RPT)])
    for j in range(CHUNK // 16):
      ones_v[pl.ds(j * 16, 16)] = ones16
    def zdfill(i, carry):
      zdeg_v[pl.ds(i * 16, 16)] = zeros16
      return carry
    lax.fori_loop(0, RPT // 16, zdfill, 0)
    def zfill(r, carry):
      for j in range(half // 16):
        rows_v[0, r, pl.ds(j * 16, 16)] = zeros16
      return carry
    lax.fori_loop(0, CHUNK, zfill, 0)
    def zinit(jj, carry):
      pltpu.sync_copy(rows_v.at[0],
                      agg_sh.at[pl.ds(sid * RPT + jj * CHUNK, CHUNK)])
      return carry
    lax.fori_loop(0, RPT // CHUNK, zinit, 0)
    pltpu.sync_copy(zdeg_v, deg_sh.at[pl.ds(sid * RPT, RPT)])
    plsc.subcore_barrier()

    def run_pipeline(p_hbm, deg_slots):
      def gsrc(b):
        return p_hbm if b in HBM_SLOTS else p_sh
      def emit_group(g, ksl):
        nsl = (ksl + 1) % NSLOT
        @pl.when(g + 1 < ngrp)
        def _prefetch():
          off = sid * NCHUNK + (g + 1) * NB
          pltpu.async_copy(src_hbm.at[pl.ds(off, NB)], sidx_v.at[nsl],
                           isems[nsl])
          pltpu.async_copy(dst_hbm.at[pl.ds(off, NB)], didx_v.at[nsl],
                           isemd[nsl])
        pltpu.make_async_copy(src_hbm.at[pl.ds(sid * NCHUNK, NB)],
                              sidx_v.at[ksl], isems[ksl]).wait()
        pltpu.make_async_copy(dst_hbm.at[pl.ds(sid * NCHUNK, NB)],
                              didx_v.at[ksl], isemd[ksl]).wait()
        for b in range(NB):
          @pl.when(g > 0)
          def _wait_prev():
            pltpu.make_async_copy(rows_v.at[b], agg_sh.at[didx_v.at[ksl, 0]],
                                  ssem[b]).wait()
            if b in deg_slots:
              pltpu.make_async_copy(ones_v, deg_sh.at[didx_v.at[ksl, 0]],
                                    dsem[b]).wait()
          pltpu.async_copy(gsrc(b).at[sidx_v.at[ksl, b]], rows_v.at[b],
                           gsem[b])
        for b in range(NB):
          pltpu.make_async_copy(gsrc(b).at[sidx_v.at[ksl, b]], rows_v.at[b],
                                gsem[b]).wait()
          pltpu.async_copy(rows_v.at[b], agg_sh.at[didx_v.at[ksl, b]],
                           ssem[b], add=True)
          if b in deg_slots:
            pltpu.async_copy(ones_v, deg_sh.at[didx_v.at[ksl, b]], dsem[b],
                             add=True)
      def gquad(gg, carry):
        for k in range(NSLOT):
          emit_group(gg * NSLOT + k, k)
        return carry
      lax.fori_loop(0, ngrp // NSLOT, gquad, 0)
      for b in range(NB):
        pltpu.make_async_copy(rows_v.at[b], agg_sh.at[didx_v.at[0, 0]],
                              ssem[b]).wait()
        if b in deg_slots:
          pltpu.make_async_copy(ones_v, deg_sh.at[didx_v.at[0, 0]],
                                dsem[b]).wait()

    @pl.when(cid == 0)
    def _core0():
      run_pipeline(pl_hbm, DEG_SLOTS_0)
    @pl.when(cid == 1)
    def _core1():
      run_pipeline(pr_hbm, DEG_SLOTS_1)
    plsc.subcore_barrier()

    @pl.when(cid == 0)
    def _wb0():
      pltpu.sync_copy(agg_sh.at[pl.ds(sid * RPT, RPT)],
                      agg_hbm.at[pl.ds(sid * RPT, RPT), pl.ds(0, half)])
      pltpu.sync_copy(deg_sh.at[pl.ds(sid * RPT, RPT)],
                      deg_hbm.at[0, pl.ds(sid * RPT, RPT)])
    @pl.when(cid == 1)
    def _wb1():
      pltpu.sync_copy(agg_sh.at[pl.ds(sid * RPT, RPT)],
                      agg_hbm.at[pl.ds(sid * RPT, RPT), pl.ds(half, half)])
      pltpu.sync_copy(deg_sh.at[pl.ds(sid * RPT, RPT)],
                      deg_hbm.at[1, pl.ds(sid * RPT, RPT)])

  return agg_kernel


_ROWS = 1000


def _mm_in_kernel(x_ref, ws_ref, wn_ref, b_ref, s_ref, pl_ref, pr_ref):
  half = pl_ref.shape[-1]
  xb = x_ref[...]
  s_ref[...] = (jnp.dot(xb, ws_ref[...], preferred_element_type=jnp.float32)
                + b_ref[...])
  pl_ref[...] = jnp.dot(xb, wn_ref[:, 0:half],
                        preferred_element_type=jnp.float32)
  pr_ref[...] = jnp.dot(xb, wn_ref[:, half:2 * half],
                        preferred_element_type=jnp.float32)


def _tc_project_in(x, W_self, W_neigh, b):
  d_in, d_out = W_self.shape
  half = d_out // 2
  grid = N // _ROWS
  return pl.pallas_call(
      _mm_in_kernel,
      grid=(grid,),
      in_specs=[
          pl.BlockSpec((_ROWS, d_in), lambda i: (i, 0)),
          pl.BlockSpec((d_in, d_out), lambda i: (0, 0)),
          pl.BlockSpec((d_in, d_out), lambda i: (0, 0)),
          pl.BlockSpec((1, d_out), lambda i: (0, 0)),
      ],
      out_specs=[
          pl.BlockSpec((_ROWS, d_out), lambda i: (i, 0)),
          pl.BlockSpec((_ROWS, half), lambda i: (i, 0)),
          pl.BlockSpec((_ROWS, half), lambda i: (i, 0)),
      ],
      out_shape=[
          jax.ShapeDtypeStruct((N, d_out), jnp.float32),
          jax.ShapeDtypeStruct((NPAD, half), jnp.float32),
          jax.ShapeDtypeStruct((NPAD, half), jnp.float32),
      ],
  )(x, W_self, W_neigh, b.reshape(1, d_out))


def _combine_mm_kernel(s_ref, agg_ref, deg_ref, ws_ref, wn_ref, b_ref,
                       s_ref_o, pl_ref, pr_ref):
  half = pl_ref.shape[-1]
  d = deg_ref[:, 0:1] + deg_ref[:, 1:2]
  h = s_ref[...] + agg_ref[...] * (1.0 / jnp.maximum(d, 1.0))
  h = jnp.maximum(h, 0.0)
  s_ref_o[...] = (jnp.dot(h, ws_ref[...], preferred_element_type=jnp.float32)
                  + b_ref[...])
  pl_ref[...] = jnp.dot(h, wn_ref[:, 0:half],
                        preferred_element_type=jnp.float32)
  pr_ref[...] = jnp.dot(h, wn_ref[:, half:2 * half],
                        preferred_element_type=jnp.float32)


def _tc_combine_project(s, agg, deg2, W_self, W_neigh, b):
  d_in, d_out = W_self.shape
  half = d_out // 2
  grid = N // _ROWS
  return pl.pallas_call(
      _combine_mm_kernel,
      grid=(grid,),
      in_specs=[
          pl.BlockSpec((_ROWS, d_in), lambda i: (i, 0)),
          pl.BlockSpec((_ROWS, d_in), lambda i: (i, 0)),
          pl.BlockSpec((_ROWS, 2), lambda i: (i, 0)),
          pl.BlockSpec((d_in, d_out), lambda i: (0, 0)),
          pl.BlockSpec((d_in, d_out), lambda i: (0, 0)),
          pl.BlockSpec((1, d_out), lambda i: (0, 0)),
      ],
      out_specs=[
          pl.BlockSpec((_ROWS, d_out), lambda i: (i, 0)),
          pl.BlockSpec((_ROWS, half), lambda i: (i, 0)),
          pl.BlockSpec((_ROWS, half), lambda i: (i, 0)),
      ],
      out_shape=[
          jax.ShapeDtypeStruct((N, d_out), jnp.float32),
          jax.ShapeDtypeStruct((NPAD, half), jnp.float32),
          jax.ShapeDtypeStruct((NPAD, half), jnp.float32),
      ],
  )(s, agg, deg2, W_self, W_neigh, b.reshape(1, d_out))


def _final_kernel(s_ref, agg_ref, deg_ref, o_ref):
  d = deg_ref[:, 0:1] + deg_ref[:, 1:2]
  o_ref[...] = s_ref[...] + agg_ref[...] * (1.0 / jnp.maximum(d, 1.0))


def _tc_final(s, agg, deg2, d_out):
  grid = N // _ROWS
  return pl.pallas_call(
      _final_kernel,
      grid=(grid,),
      in_specs=[
          pl.BlockSpec((_ROWS, d_out), lambda i: (i, 0)),
          pl.BlockSpec((_ROWS, d_out), lambda i: (i, 0)),
          pl.BlockSpec((_ROWS, 2), lambda i: (i, 0)),
      ],
      out_specs=pl.BlockSpec((_ROWS, d_out), lambda i: (i, 0)),
      out_shape=jax.ShapeDtypeStruct((N, d_out), jnp.float32),
  )(s, agg, deg2)


_sc_agg_128 = _make_sc_aggregate(128)
_sc_agg_64 = _make_sc_aggregate(64)


@jax.jit
def kernel(x, edge_index0, edge_index1, W_self0, W_neigh0, b0,
           W_self1, W_neigh1, b1):
  src0 = edge_index0[0].reshape(TOT_CHUNKS, CHUNK)
  dst0 = edge_index0[1].reshape(TOT_CHUNKS, CHUNK)
  src1 = edge_index1[0].reshape(TOT_CHUNKS, CHUNK)
  dst1 = edge_index1[1].reshape(TOT_CHUNKS, CHUNK)

  s0, p0l, p0r = _tc_project_in(x, W_self0, W_neigh0, b0)
  agg0, deg0 = _sc_agg_128(p0l, p0r, src0, dst0)
  s1, p1l, p1r = _tc_combine_project(s0, agg0, deg0.T, W_self1, W_neigh1, b1)
  agg1, deg1 = _sc_agg_64(p1l, p1r, src1, dst1)
  out = _tc_final(s1, agg1, deg1.T, 64)
  return out


# speedup vs baseline: 1.0144x; 1.0144x over previous
import functools

import jax
import jax.numpy as jnp
from jax import lax
from jax.experimental import pallas as pl
from jax.experimental.pallas import tpu as pltpu
from jax.experimental.pallas import tpu_sc as plsc

N = 10000
E = 320000
NPAD = 10240

NUM_SUBCORES = 16
CHUNK = 128
TOT_CHUNKS = E // CHUNK
NCHUNK = 160
NB = 5
NGROUP = NCHUNK // NB
NSLOT = 4
HBM_SLOTS = ()
DEG_SLOTS_0 = (0, 1, 2, 3, 4)
DEG_SLOTS_1 = ()
RPT = NPAD // 16


def _make_sc_aggregate(D):
  half = D // 2
  mesh = plsc.VectorSubcoreMesh(core_axis_name="c", subcore_axis_name="s")

  @functools.partial(
      pl.kernel,
      out_type=(
          jax.ShapeDtypeStruct((NPAD, D), jnp.float32),
          jax.ShapeDtypeStruct((2, NPAD), jnp.float32),
      ),
      mesh=mesh,
      compiler_params=pltpu.CompilerParams(use_tc_tiling_on_sc=False),
      scratch_types=[
          pltpu.VMEM((NSLOT, NB, CHUNK), jnp.int32),
          pltpu.VMEM((NSLOT, NB, CHUNK), jnp.int32),
          pltpu.VMEM((NB, CHUNK, half), jnp.float32),
          pltpu.VMEM((CHUNK,), jnp.float32),
          pltpu.VMEM((RPT,), jnp.float32),
          pltpu.VMEM_SHARED((NPAD, half), jnp.float32),
          pltpu.VMEM_SHARED((NPAD, half), jnp.float32),
          pltpu.VMEM_SHARED((NPAD,), jnp.float32),
      ] + [pltpu.SemaphoreType.DMA] * (3 * NB + 2 * NSLOT),
  )
  def agg_kernel(pl_hbm, pr_hbm, src_hbm, dst_hbm, agg_hbm, deg_hbm,
                 sidx_v, didx_v, rows_v, ones_v, zdeg_v, p_sh, agg_sh, deg_sh,
                 *sems):
    gsem = sems[:NB]
    ssem = sems[NB:2 * NB]
    dsem = sems[2 * NB:3 * NB]
    isems = sems[3 * NB:3 * NB + NSLOT]
    isemd = sems[3 * NB + NSLOT:]
    cid = lax.axis_index("c")
    sid = lax.axis_index("s")
    ngrp = jnp.minimum(NCHUNK, TOT_CHUNKS - sid * NCHUNK) // NB

    zeros16 = jnp.zeros((16,), jnp.float32)
    ones16 = jnp.ones((16,), jnp.float32)

    pltpu.async_copy(src_hbm.at[pl.ds(sid * NCHUNK, NB)], sidx_v.at[0],
                     isems[0])
    pltpu.async_copy(dst_hbm.at[pl.ds(sid * NCHUNK, NB)], didx_v.at[0],
                     isemd[0])
    @pl.when(cid == 0)
    def _stage0():
      pltpu.sync_copy(pl_hbm.at[pl.ds(sid * RPT, RPT)],
                      p_sh.at[pl.ds(sid * RPT, RPT)])
    @pl.when(cid == 1)
    def _stage1():
      pltpu.sync_copy(pr_hbm.at[pl.ds(sid * RPT, RPT)],
                      p_sh.at[pl.ds(sid * RPT, RPT)])
    for j in range(CHUNK // 16):
      ones_v[pl.ds(j * 16, 16)] = ones16
    def zdfill(i, carry):
      zdeg_v[pl.ds(i * 16, 16)] = zeros16
      return carry
    lax.fori_loop(0, RPT // 16, zdfill, 0)
    def zfill(r, carry):
      for j in range(half // 16):
        rows_v[0, r, pl.ds(j * 16, 16)] = zeros16
      return carry
    lax.fori_loop(0, CHUNK, zfill, 0)
    def zinit(jj, carry):
      pltpu.sync_copy(rows_v.at[0],
                      agg_sh.at[pl.ds(sid * RPT + jj * CHUNK, CHUNK)])
      return carry
    lax.fori_loop(0, RPT // CHUNK, zinit, 0)
    pltpu.sync_copy(zdeg_v, deg_sh.at[pl.ds(sid * RPT, RPT)])
    plsc.subcore_barrier()

    def run_pipeline(p_hbm, deg_slots):
      def gsrc(b):
        return p_hbm if b in HBM_SLOTS else p_sh
      def emit_group(g, ksl):
        nsl = (ksl + 1) % NSLOT
        @pl.when(g + 1 < ngrp)
        def _prefetch():
          off = sid * NCHUNK + (g + 1) * NB
          pltpu.async_copy(src_hbm.at[pl.ds(off, NB)], sidx_v.at[nsl],
                           isems[nsl])
          pltpu.async_copy(dst_hbm.at[pl.ds(off, NB)], didx_v.at[nsl],
                           isemd[nsl])
        pltpu.make_async_copy(src_hbm.at[pl.ds(sid * NCHUNK, NB)],
                              sidx_v.at[ksl], isems[ksl]).wait()
        pltpu.make_async_copy(dst_hbm.at[pl.ds(sid * NCHUNK, NB)],
                              didx_v.at[ksl], isemd[ksl]).wait()
        for b in range(NB):
          @pl.when(g > 0)
          def _wait_prev():
            pltpu.make_async_copy(rows_v.at[b], agg_sh.at[didx_v.at[ksl, 0]],
                                  ssem[b]).wait()
            if b in deg_slots:
              pltpu.make_async_copy(ones_v, deg_sh.at[didx_v.at[ksl, 0]],
                                    dsem[b]).wait()
          pltpu.async_copy(gsrc(b).at[sidx_v.at[ksl, b]], rows_v.at[b],
                           gsem[b])
        for b in range(NB):
          pltpu.make_async_copy(gsrc(b).at[sidx_v.at[ksl, b]], rows_v.at[b],
                                gsem[b]).wait()
          pltpu.async_copy(rows_v.at[b], agg_sh.at[didx_v.at[ksl, b]],
                           ssem[b], add=True)
          if b in deg_slots:
            pltpu.async_copy(ones_v, deg_sh.at[didx_v.at[ksl, b]], dsem[b],
                             add=True)
      def gquad(gg, carry):
        for k in range(NSLOT):
          emit_group(gg * NSLOT + k, k)
        return carry
      lax.fori_loop(0, ngrp // NSLOT, gquad, 0)
      for b in range(NB):
        pltpu.make_async_copy(rows_v.at[b], agg_sh.at[didx_v.at[0, 0]],
                              ssem[b]).wait()
        if b in deg_slots:
          pltpu.make_async_copy(ones_v, deg_sh.at[didx_v.at[0, 0]],
                                dsem[b]).wait()

    @pl.when(cid == 0)
    def _core0():
      run_pipeline(pl_hbm, DEG_SLOTS_0)
    @pl.when(cid == 1)
    def _core1():
      run_pipeline(pr_hbm, DEG_SLOTS_1)
    plsc.subcore_barrier()

    @pl.when(cid == 0)
    def _wb0():
      pltpu.sync_copy(agg_sh.at[pl.ds(sid * RPT, RPT)],
                      agg_hbm.at[pl.ds(sid * RPT, RPT), pl.ds(0, half)])
      pltpu.sync_copy(deg_sh.at[pl.ds(sid * RPT, RPT)],
                      deg_hbm.at[0, pl.ds(sid * RPT, RPT)])
    @pl.when(cid == 1)
    def _wb1():
      pltpu.sync_copy(agg_sh.at[pl.ds(sid * RPT, RPT)],
                      agg_hbm.at[pl.ds(sid * RPT, RPT), pl.ds(half, half)])
      pltpu.sync_copy(zdeg_v, deg_hbm.at[1, pl.ds(sid * RPT, RPT)])

  return agg_kernel


_ROWS = 1000


def _mm_p_kernel(x_ref, wn_ref, pl_ref, pr_ref):
  half = pl_ref.shape[-1]
  xb = x_ref[...]
  pl_ref[...] = jnp.dot(xb, wn_ref[:, 0:half],
                        preferred_element_type=jnp.float32)
  pr_ref[...] = jnp.dot(xb, wn_ref[:, half:2 * half],
                        preferred_element_type=jnp.float32)


def _tc_project_p(x, W_neigh):
  d_in, d_out = W_neigh.shape
  half = d_out // 2
  grid = N // _ROWS
  return pl.pallas_call(
      _mm_p_kernel,
      grid=(grid,),
      in_specs=[
          pl.BlockSpec((_ROWS, d_in), lambda i: (i, 0)),
          pl.BlockSpec((d_in, d_out), lambda i: (0, 0)),
      ],
      out_specs=[
          pl.BlockSpec((_ROWS, half), lambda i: (i, 0)),
          pl.BlockSpec((_ROWS, half), lambda i: (i, 0)),
      ],
      out_shape=[
          jax.ShapeDtypeStruct((NPAD, half), jnp.float32),
          jax.ShapeDtypeStruct((NPAD, half), jnp.float32),
      ],
  )(x, W_neigh)


def _mm_s_kernel(x_ref, ws_ref, b_ref, s_ref):
  s_ref[...] = (jnp.dot(x_ref[...], ws_ref[...],
                        preferred_element_type=jnp.float32) + b_ref[...])


def _tc_project_s(x, W_self, b):
  d_in, d_out = W_self.shape
  grid = N // _ROWS
  return pl.pallas_call(
      _mm_s_kernel,
      grid=(grid,),
      in_specs=[
          pl.BlockSpec((_ROWS, d_in), lambda i: (i, 0)),
          pl.BlockSpec((d_in, d_out), lambda i: (0, 0)),
          pl.BlockSpec((1, d_out), lambda i: (0, 0)),
      ],
      out_specs=pl.BlockSpec((_ROWS, d_out), lambda i: (i, 0)),
      out_shape=jax.ShapeDtypeStruct((N, d_out), jnp.float32),
  )(x, W_self, b.reshape(1, d_out))


def _combine_mm_kernel(s_ref, agg_ref, deg_ref, wn_ref, h_ref, pl_ref,
                       pr_ref):
  half = pl_ref.shape[-1]
  d = deg_ref[:, 0:1] + deg_ref[:, 1:2]
  h = s_ref[...] + agg_ref[...] * (1.0 / jnp.maximum(d, 1.0))
  h = jnp.maximum(h, 0.0)
  h_ref[...] = h
  pl_ref[...] = jnp.dot(h, wn_ref[:, 0:half],
                        preferred_element_type=jnp.float32)
  pr_ref[...] = jnp.dot(h, wn_ref[:, half:2 * half],
                        preferred_element_type=jnp.float32)


def _tc_combine_project(s, agg, deg2, W_neigh):
  d_in, d_out = W_neigh.shape
  half = d_out // 2
  grid = N // _ROWS
  return pl.pallas_call(
      _combine_mm_kernel,
      grid=(grid,),
      in_specs=[
          pl.BlockSpec((_ROWS, d_in), lambda i: (i, 0)),
          pl.BlockSpec((_ROWS, d_in), lambda i: (i, 0)),
          pl.BlockSpec((_ROWS, 2), lambda i: (i, 0)),
          pl.BlockSpec((d_in, d_out), lambda i: (0, 0)),
      ],
      out_specs=[
          pl.BlockSpec((_ROWS, d_in), lambda i: (i, 0)),
          pl.BlockSpec((_ROWS, half), lambda i: (i, 0)),
          pl.BlockSpec((_ROWS, half), lambda i: (i, 0)),
      ],
      out_shape=[
          jax.ShapeDtypeStruct((N, d_in), jnp.float32),
          jax.ShapeDtypeStruct((NPAD, half), jnp.float32),
          jax.ShapeDtypeStruct((NPAD, half), jnp.float32),
      ],
  )(s, agg, deg2, W_neigh)


def _final_kernel(s_ref, agg_ref, deg_ref, o_ref):
  d = deg_ref[:, 0:1] + deg_ref[:, 1:2]
  o_ref[...] = s_ref[...] + agg_ref[...] * (1.0 / jnp.maximum(d, 1.0))


def _tc_final(s, agg, deg2, d_out):
  grid = N // _ROWS
  return pl.pallas_call(
      _final_kernel,
      grid=(grid,),
      in_specs=[
          pl.BlockSpec((_ROWS, d_out), lambda i: (i, 0)),
          pl.BlockSpec((_ROWS, d_out), lambda i: (i, 0)),
          pl.BlockSpec((_ROWS, 2), lambda i: (i, 0)),
      ],
      out_specs=pl.BlockSpec((_ROWS, d_out), lambda i: (i, 0)),
      out_shape=jax.ShapeDtypeStruct((N, d_out), jnp.float32),
  )(s, agg, deg2)


_sc_agg_128 = _make_sc_aggregate(128)
_sc_agg_64 = _make_sc_aggregate(64)


@jax.jit
def kernel(x, edge_index0, edge_index1, W_self0, W_neigh0, b0,
           W_self1, W_neigh1, b1):
  src0 = edge_index0[0].reshape(TOT_CHUNKS, CHUNK)
  dst0 = edge_index0[1].reshape(TOT_CHUNKS, CHUNK)
  src1 = edge_index1[0].reshape(TOT_CHUNKS, CHUNK)
  dst1 = edge_index1[1].reshape(TOT_CHUNKS, CHUNK)

  p0l, p0r = _tc_project_p(x, W_neigh0)
  agg0, deg0 = _sc_agg_128(p0l, p0r, src0, dst0)
  s0 = _tc_project_s(x, W_self0, b0)
  h1, p1l, p1r = _tc_combine_project(s0, agg0, deg0.T, W_neigh1)
  agg1, deg1 = _sc_agg_64(p1l, p1r, src1, dst1)
  s1 = _tc_project_s(h1, W_self1, b1)
  out = _tc_final(s1, agg1, deg1.T, 64)
  return out
